# baseline (speedup 1.0000x reference)
import functools

import jax
import jax.numpy as jnp
from jax import lax
from jax.experimental import pallas as pl
from jax.experimental.pallas import tpu as pltpu
from jax.experimental.pallas import tpu_sc as plsc

_B = 16
_K = 128
_LANES = 16
_NEG_INF_BITS = -8388608


def _sim_body(n_rows, tile_n, x1_ref, x2_ref, w_ref, b_ref, batch_ref,
              o_ref, cnt_ref):
    dims = (((1,), (0,)), ((), ()))
    e1 = lax.dot_general(x1_ref[...], w_ref[...], dims,
                         preferred_element_type=jnp.float32) + b_ref[...]
    e2 = lax.dot_general(x2_ref[...], w_ref[...], dims,
                         preferred_element_type=jnp.float32) + b_ref[...]
    s = e1 - e2 + 1e-6
    val = jnp.sqrt(jnp.sum(s * s, axis=1, keepdims=True))
    row0 = pl.program_id(0) * tile_n
    rid = row0 + lax.broadcasted_iota(jnp.int32, (tile_n, 1), 0)
    o_ref[...] = jnp.where(rid < n_rows, val, 0.0)
    @pl.when(pl.program_id(0) == 0)
    def _init():
        cnt_ref[...] = jnp.zeros_like(cnt_ref)
    b = jnp.where(rid < n_rows, batch_ref[...], _B)
    gid = lax.broadcasted_iota(jnp.int32, (1, _B), 1)
    cmp = (b == gid).astype(jnp.int32)
    cnt_ref[...] = cnt_ref[...] + jnp.sum(cmp, axis=0, keepdims=True)


def _pairwise_sim(x1, x2, w_emb, b_emb, batch32, tile_n=1024):
    n, d = x1.shape
    n_tiles = pl.cdiv(n, tile_n)
    n_pad = n_tiles * tile_n
    sim2d, counts = pl.pallas_call(
        functools.partial(_sim_body, n, tile_n),
        grid=(n_tiles,),
        in_specs=[
            pl.BlockSpec((tile_n, d), lambda i: (i, 0)),
            pl.BlockSpec((tile_n, d), lambda i: (i, 0)),
            pl.BlockSpec((d, d), lambda i: (0, 0)),
            pl.BlockSpec((1, d), lambda i: (0, 0)),
            pl.BlockSpec((tile_n, 1), lambda i: (i, 0)),
        ],
        out_specs=[
            pl.BlockSpec((tile_n, 1), lambda i: (i, 0)),
            pl.BlockSpec((1, _B), lambda i: (0, 0)),
        ],
        out_shape=[
            jax.ShapeDtypeStruct((n_pad, 1), jnp.float32),
            jax.ShapeDtypeStruct((1, _B), jnp.int32),
        ],
    )(x1, x2, w_emb, b_emb.reshape(1, d), batch32.reshape(n, 1))
    return sim2d.reshape(n_pad), counts.reshape(_B)


def _sc_topk_body(n_pad, sim_hbm, cnt_hbm, out_hbm,
                  sim_v, cnt_v, out_v, key_v):
    c = lax.axis_index("c")
    s = lax.axis_index("s")
    g = s * 2 + c

    @pl.when(g < _B)
    def _work():
        pltpu.sync_copy(sim_hbm, sim_v)
        pltpu.sync_copy(cnt_hbm, cnt_v)

        lanes = lax.iota(jnp.int32, _LANES)
        cnt = cnt_v[...]
        en_vec = plsc.cumsum(cnt)
        st_vec = en_vec - cnt
        start = jnp.int32(0)
        end = jnp.int32(0)
        for l in range(_B):
            start = jnp.where(g == l, st_vec[l], start)
            end = jnp.where(g == l, en_vec[l], end)
        n_g = end - start
        t0 = start // _LANES
        nv = (end + _LANES - 1) // _LANES - t0

        def compact(j, carry):
            base = (t0 + j) * _LANES
            k = plsc.bitcast(sim_v[pl.ds(base, _LANES)], jnp.int32)
            gl = lanes + base
            m = (gl >= start) & (gl < end)
            key_v[pl.ds(j * _LANES, _LANES)] = jnp.where(m, k, -1)
            return carry
        lax.fori_loop(0, nv, compact, 0)

        def count_ge(th):
            thv = jnp.full((_LANES,), th, dtype=jnp.int32)
            def cb(j, acc):
                kv = key_v[pl.ds(j * _LANES, _LANES)]
                return acc + plsc.all_reduce_population_count(kv >= thv)
            acc = lax.fori_loop(0, nv, cb, jnp.zeros((_LANES,), jnp.int32))
            return acc[0]

        def bs(i, lohi):
            lo, hi = lohi
            mid = lo + ((hi - lo) // 2) + ((hi - lo) & 1)
            pred = count_ge(mid) >= _K
            return (jnp.where(pred, mid, lo), jnp.where(pred, hi, mid - 1))
        v_k, _ = lax.fori_loop(0, 31, bs, (jnp.int32(0), jnp.int32(0x7FFFFFFF)))

        small = n_g < _K
        ext_th = jnp.where(small, jnp.int32(0), v_k + 1)
        fill_bits = jnp.where(small, _NEG_INF_BITS, v_k)
        fill_v = plsc.bitcast(jnp.full((_LANES,), fill_bits, dtype=jnp.int32),
                              jnp.float32)
        for r in range(_K // _LANES):
            out_v[pl.ds(r * _LANES, _LANES)] = fill_v

        thv = jnp.full((_LANES,), ext_th, dtype=jnp.int32)
        def extract(j, off):
            kv = key_v[pl.ds(j * _LANES, _LANES)]
            m = kv >= thv
            cum = plsc.cumsum(m.astype(jnp.int32))
            pos = off + cum - 1
            plsc.store_scatter(out_v, [pos], plsc.bitcast(kv, jnp.float32),
                               mask=m)
            return off + plsc.all_reduce_population_count(m)[0]
        lax.fori_loop(0, nv, extract, jnp.int32(0))

        pltpu.sync_copy(out_v, out_hbm.at[g])


def _sc_topk(sim, counts):
    n_pad = sim.shape[0]
    mesh = plsc.VectorSubcoreMesh(core_axis_name="c", subcore_axis_name="s")
    return pl.kernel(
        functools.partial(_sc_topk_body, n_pad),
        out_type=jax.ShapeDtypeStruct((_B, _K), jnp.float32),
        mesh=mesh,
        scratch_types=[
            pltpu.VMEM((n_pad,), jnp.float32),
            pltpu.VMEM((_LANES,), jnp.int32),
            pltpu.VMEM((_K,), jnp.float32),
            pltpu.VMEM((n_pad,), jnp.int32),
        ],
        compiler_params=pltpu.CompilerParams(needs_layout_passes=False),
    )(sim, counts)


def _mlp_body(cand_ref, w1_ref, b1_ref, w2_ref, b2_ref, o_ref):
    x = cand_ref[...]
    lane = lax.broadcasted_iota(jnp.int32, (_B, _K), 1)
    rank = jnp.zeros((_B, _K), jnp.int32)
    rv = x
    for d in range(1, _K):
        rv = jnp.roll(rv, -1, axis=1)
        gt = rv > x
        tie = (rv == x) & (lane >= _K - d)
        rank = rank + jnp.where(gt | tie, 1, 0)
    xs = jnp.where(rank == lane, x, 0.0)
    rv, rk = x, rank
    for d in range(1, _K):
        rv = jnp.roll(rv, -1, axis=1)
        rk = jnp.roll(rk, -1, axis=1)
        xs = xs + jnp.where(rk == lane, rv, 0.0)
    h = jnp.maximum(
        lax.dot_general(xs, w1_ref[...], (((1,), (0,)), ((), ())),
                        preferred_element_type=jnp.float32) + b1_ref[...], 0.0)
    o_ref[...] = lax.dot_general(h, w2_ref[...], (((1,), (0,)), ((), ())),
                                 preferred_element_type=jnp.float32) + b2_ref[...]


def _sort_mlp(cand, w1, b1, w2, b2):
    return pl.pallas_call(
        _mlp_body,
        out_shape=jax.ShapeDtypeStruct((_B, 1), jnp.float32),
    )(cand, w1, b1.reshape(1, -1), w2, b2.reshape(1, 1))


def kernel(x1, x2, batch, W_emb, b_emb, W1, b1, W2, b2):
    batch32 = batch.astype(jnp.int32)
    sim, counts = _pairwise_sim(x1, x2, W_emb, b_emb, batch32)
    cand = _sc_topk(sim, counts)
    return _sort_mlp(cand, W1, b1, W2, b2)


# speedup vs baseline: 1.0625x; 1.0625x over previous
import functools

import jax
import jax.numpy as jnp
from jax import lax
from jax.experimental import pallas as pl
from jax.experimental.pallas import tpu as pltpu
from jax.experimental.pallas import tpu_sc as plsc

_B = 16
_K = 128
_LANES = 16
_NEG_INF_BITS = -8388608


def _sim_body(n_rows, tile_n, x1_ref, x2_ref, w_ref, b_ref, batch_ref,
              o_ref, cnt_ref):
    dims = (((1,), (0,)), ((), ()))
    e1 = lax.dot_general(x1_ref[...], w_ref[...], dims,
                         preferred_element_type=jnp.float32) + b_ref[...]
    e2 = lax.dot_general(x2_ref[...], w_ref[...], dims,
                         preferred_element_type=jnp.float32) + b_ref[...]
    s = e1 - e2 + 1e-6
    val = jnp.sqrt(jnp.sum(s * s, axis=1, keepdims=True))
    row0 = pl.program_id(0) * tile_n
    rid = row0 + lax.broadcasted_iota(jnp.int32, (tile_n, 1), 0)
    o_ref[...] = jnp.where(rid < n_rows, val, 0.0)
    @pl.when(pl.program_id(0) == 0)
    def _init():
        cnt_ref[...] = jnp.zeros_like(cnt_ref)
    b = jnp.where(rid < n_rows, batch_ref[...], _B)
    gid = lax.broadcasted_iota(jnp.int32, (1, _B), 1)
    cmp = (b == gid).astype(jnp.int32)
    cnt_ref[...] = cnt_ref[...] + jnp.sum(cmp, axis=0, keepdims=True)


def _pairwise_sim(x1, x2, w_emb, b_emb, batch32, tile_n=1024):
    n, d = x1.shape
    n_tiles = pl.cdiv(n, tile_n)
    n_pad = n_tiles * tile_n
    sim2d, counts = pl.pallas_call(
        functools.partial(_sim_body, n, tile_n),
        grid=(n_tiles,),
        in_specs=[
            pl.BlockSpec((tile_n, d), lambda i: (i, 0)),
            pl.BlockSpec((tile_n, d), lambda i: (i, 0)),
            pl.BlockSpec((d, d), lambda i: (0, 0)),
            pl.BlockSpec((1, d), lambda i: (0, 0)),
            pl.BlockSpec((tile_n, 1), lambda i: (i, 0)),
        ],
        out_specs=[
            pl.BlockSpec((tile_n, 1), lambda i: (i, 0)),
            pl.BlockSpec((1, _B), lambda i: (0, 0)),
        ],
        out_shape=[
            jax.ShapeDtypeStruct((n_pad, 1), jnp.float32),
            jax.ShapeDtypeStruct((1, _B), jnp.int32),
        ],
    )(x1, x2, w_emb, b_emb.reshape(1, d), batch32.reshape(n, 1))
    return sim2d.reshape(n_pad), counts.reshape(_B)


def _rev(v):
    return lax.rev(v, dimensions=(0,))


def _vsort_desc(v):
    return _rev(lax.sort(v))


def _merge_clean(lst):
    n = len(lst)
    if n == 1:
        return [_vsort_desc(lst[0])]
    half = n // 2
    hi = [jnp.maximum(lst[i], lst[i + half]) for i in range(half)]
    lo = [jnp.minimum(lst[i], lst[i + half]) for i in range(half)]
    return _merge_clean(hi) + _merge_clean(lo)


def _merge_desc(a, b):
    return _merge_clean(a + [_rev(v) for v in reversed(b)])


def _sc_topk_body(n_pad, sim_hbm, cnt_hbm, w1_hbm, b1_hbm, w2_hbm, b2_hbm,
                  out_hbm, sim_v, cnt_v, out_v, key_v, w1_v, b1_v, w2_v,
                  b2_v, res_v):
    c = lax.axis_index("c")
    s = lax.axis_index("s")
    g = s * 2 + c

    @pl.when(g < _B)
    def _work():
        pltpu.sync_copy(sim_hbm, sim_v)
        pltpu.sync_copy(cnt_hbm, cnt_v)
        pltpu.sync_copy(w1_hbm, w1_v)
        pltpu.sync_copy(b1_hbm, b1_v)
        pltpu.sync_copy(w2_hbm, w2_v)
        pltpu.sync_copy(b2_hbm, b2_v)

        lanes = lax.iota(jnp.int32, _LANES)
        cnt = cnt_v[...]
        en_vec = plsc.cumsum(cnt)
        st_vec = en_vec - cnt
        start = jnp.int32(0)
        end = jnp.int32(0)
        for l in range(_B):
            start = jnp.where(g == l, st_vec[l], start)
            end = jnp.where(g == l, en_vec[l], end)
        n_g = end - start
        t0 = start // _LANES
        nv = (end + _LANES - 1) // _LANES - t0

        def compact(j, carry):
            base = (t0 + j) * _LANES
            k = plsc.bitcast(sim_v[pl.ds(base, _LANES)], jnp.int32)
            gl = lanes + base
            m = (gl >= start) & (gl < end)
            key_v[pl.ds(j * _LANES, _LANES)] = jnp.where(m, k, -1)
            return carry
        lax.fori_loop(0, nv, compact, 0)

        def count_ge(th):
            thv = jnp.full((_LANES,), th, dtype=jnp.int32)
            def cb(j, acc):
                kv = key_v[pl.ds(j * _LANES, _LANES)]
                return acc + plsc.all_reduce_population_count(kv >= thv)
            acc = lax.fori_loop(0, nv, cb, jnp.zeros((_LANES,), jnp.int32))
            return acc[0]

        def bs(i, lohi):
            lo, hi = lohi
            mid = lo + ((hi - lo) // 2) + ((hi - lo) & 1)
            pred = count_ge(mid) >= _K
            return (jnp.where(pred, mid, lo), jnp.where(pred, hi, mid - 1))
        v_k, _ = lax.fori_loop(0, 31, bs, (jnp.int32(0), jnp.int32(0x7FFFFFFF)))

        small = n_g < _K
        ext_th = jnp.where(small, jnp.int32(0), v_k + 1)
        fill_bits = jnp.where(small, _NEG_INF_BITS, v_k)
        fill_v = plsc.bitcast(jnp.full((_LANES,), fill_bits, dtype=jnp.int32),
                              jnp.float32)
        for r in range(_K // _LANES):
            out_v[pl.ds(r * _LANES, _LANES)] = fill_v

        thv = jnp.full((_LANES,), ext_th, dtype=jnp.int32)
        def extract(j, off):
            kv = key_v[pl.ds(j * _LANES, _LANES)]
            m = kv >= thv
            cum = plsc.cumsum(m.astype(jnp.int32))
            pos = off + cum - 1
            plsc.store_scatter(out_v, [pos], plsc.bitcast(kv, jnp.float32),
                               mask=m)
            return off + plsc.all_reduce_population_count(m)[0]
        lax.fori_loop(0, nv, extract, jnp.int32(0))

        runs = [[_vsort_desc(out_v[pl.ds(i * _LANES, _LANES)])]
                for i in range(_K // _LANES)]
        while len(runs) > 1:
            runs = [_merge_desc(runs[i], runs[i + 1])
                    for i in range(0, len(runs), 2)]
        xs = runs[0]

        h = b1_v[...]
        for i in range(len(xs)):
            for l in range(_LANES):
                r = i * _LANES + l
                h = h + xs[i][l] * w1_v[pl.ds(r * 16, 16)]
        h = jnp.maximum(h, 0.0)
        tot = plsc.cumsum(h * w2_v[...])[_LANES - 1] + b2_v[...][0]
        res_v[...] = jnp.full((_LANES,), tot, dtype=jnp.float32)
        pltpu.sync_copy(res_v.at[pl.ds(0, 8)], out_hbm.at[pl.ds(g * 8, 8)])


def _sc_topk_mlp(sim, counts, w1, b1, w2, b2):
    n_pad = sim.shape[0]
    h = w1.shape[1]
    b1p = b1.reshape(h)
    w2p = w2.reshape(h)
    b2p = jnp.broadcast_to(b2.reshape(1), (_LANES,))
    mesh = plsc.VectorSubcoreMesh(core_axis_name="c", subcore_axis_name="s")
    out8 = pl.kernel(
        functools.partial(_sc_topk_body, n_pad),
        out_type=jax.ShapeDtypeStruct((_B * 8,), jnp.float32),
        mesh=mesh,
        scratch_types=[
            pltpu.VMEM((n_pad,), jnp.float32),
            pltpu.VMEM((_LANES,), jnp.int32),
            pltpu.VMEM((_K,), jnp.float32),
            pltpu.VMEM((n_pad,), jnp.int32),
            pltpu.VMEM((_K * h,), jnp.float32),
            pltpu.VMEM((h,), jnp.float32),
            pltpu.VMEM((h,), jnp.float32),
            pltpu.VMEM((_LANES,), jnp.float32),
            pltpu.VMEM((_LANES,), jnp.float32),
        ],
        compiler_params=pltpu.CompilerParams(needs_layout_passes=False),
    )(sim, counts, w1.reshape(_K * h), b1p, w2p, b2p)
    return out8.reshape(_B, 8)[:, :1]


def kernel(x1, x2, batch, W_emb, b_emb, W1, b1, W2, b2):
    batch32 = batch.astype(jnp.int32)
    sim, counts = _pairwise_sim(x1, x2, W_emb, b_emb, batch32)
    return _sc_topk_mlp(sim, counts, W1, b1, W2, b2)
